# baseline (speedup 1.0000x reference)
import functools

import jax
import jax.numpy as jnp
from jax import lax
from jax.experimental import pallas as pl
from jax.experimental.pallas import tpu as pltpu
from jax.experimental.pallas import tpu_sc as plsc

_B, _S, _H = 4, 8192, 4096
_E = 64
_TOP_K = 8
_KERNEL = 4
_BS = 1024
_NJ = _S // _BS

_NW = 32
_TPW = _S // _NW
_CH = 64


def _logits_body(x_ref, w_ref, b_ref, out_ref, ytail):
    j = pl.program_id(0)
    y = jnp.dot(x_ref[0], w_ref[...], preferred_element_type=jnp.float32)

    @pl.when(j == 0)
    def _():
        ytail[...] = jnp.zeros_like(ytail)

    prev = ytail[0:3, :]
    ycat = jnp.concatenate([prev, y], axis=0)
    logits = (ycat[3:3 + _BS, 3 * _E:4 * _E]
              + ycat[2:2 + _BS, 2 * _E:3 * _E]
              + ycat[1:1 + _BS, 1 * _E:2 * _E]
              + ycat[0:_BS, 0:_E]) + b_ref[0]
    ytail[0:3, :] = y[_BS - 3:_BS, :]
    out_ref[...] = logits


def _tc_logits(hidden_states, wpack, bias2, b):
    return pl.pallas_call(
        _logits_body,
        grid=(_NJ,),
        in_specs=[
            pl.BlockSpec((1, _BS, _H), lambda j: (b, j, 0)),
            pl.BlockSpec((_H, _KERNEL * _E), lambda j: (0, 0)),
            pl.BlockSpec((1, _E), lambda j: (0, 0)),
        ],
        out_specs=pl.BlockSpec((_BS, _E), lambda j: (j, 0)),
        out_shape=jax.ShapeDtypeStruct((_S, _E), jnp.float32),
        scratch_shapes=[pltpu.VMEM((8, _KERNEL * _E), jnp.float32)],
        compiler_params=pltpu.CompilerParams(
            dimension_semantics=("arbitrary",),
        ),
    )(hidden_states, wpack, bias2)


_GDN = lax.GatherDimensionNumbers(
    offset_dims=(), collapsed_slice_dims=(0,), start_index_map=(0,))


def _route_sc_body(lgi_hbm, lgf_hbm, idx_hbm, wt_hbm, lbi, lbf, oidx, owt):
    wid = lax.axis_index("s") * 2 + lax.axis_index("c")
    base = wid * _TPW
    lane = lax.iota(jnp.int32, 16)
    mask8 = lane < 8

    def shuf(v, d):
        return lax.gather(v, (lane ^ d)[:, None], _GDN, (1,),
                          mode=lax.GatherScatterMode.PROMISE_IN_BOUNDS)

    def fold(v, op):
        for d in (8, 4, 2, 1):
            v = op(v, shuf(v, d))
        return v

    for c in range(_TPW // _CH):
        start = base + c * _CH
        pltpu.sync_copy(lgi_hbm.at[pl.ds(start * _E, _CH * _E)], lbi)
        pltpu.sync_copy(lgf_hbm.at[pl.ds(start * _E, _CH * _E)], lbf)

        def tok(t, carry):
            ky = []
            for k in range(4):
                bi = lbi[pl.ds(t * _E + k * 16, 16)]
                km = bi ^ (jnp.int32(0x7FFFFFFF) & (bi >> 31))
                ky.append((km & jnp.int32(-64)) | (63 - (lane + 16 * k)))

            tt = jnp.zeros((16,), jnp.int32)
            for s in range(_TOP_K):
                mv = jnp.maximum(jnp.maximum(ky[0], ky[1]),
                                 jnp.maximum(ky[2], ky[3]))
                kmax = fold(mv, jnp.maximum)
                tt = jnp.where(lane == s, kmax, tt)
                for k in range(4):
                    ky[k] = jnp.where(ky[k] == kmax, jnp.int32(-2 ** 31),
                                      ky[k])
            idxv = 63 - (tt & 63)
            bank = idxv >> 4
            off = idxv & 15
            l8 = jnp.zeros((16,), jnp.float32)
            for k in range(4):
                vk = lbf[pl.ds(t * _E + k * 16, 16)]
                gk = lax.gather(vk, off[:, None], _GDN, (1,),
                                mode=lax.GatherScatterMode.PROMISE_IN_BOUNDS)
                l8 = jnp.where(bank == k, gk, l8)
            m = fold(l8, jnp.maximum)
            e = jnp.exp(l8 - m)
            em = jnp.where(mask8, e, 0.0)
            den = fold(em, jnp.add)
            owt[pl.ds(t * 8, 16)] = em / den
            oidx[pl.ds(t * 8, 16)] = idxv
            return carry

        lax.fori_loop(0, _CH, tok, 0, unroll=2)
        pltpu.sync_copy(oidx.at[pl.ds(0, _CH * 8)],
                        idx_hbm.at[pl.ds(start * 8, _CH * 8)])
        pltpu.sync_copy(owt.at[pl.ds(0, _CH * 8)],
                        wt_hbm.at[pl.ds(start * 8, _CH * 8)])


_route_sc = functools.partial(
    pl.kernel,
    mesh=plsc.VectorSubcoreMesh(core_axis_name="c", subcore_axis_name="s"),
    out_type=[
        jax.ShapeDtypeStruct((_S * 8,), jnp.int32),
        jax.ShapeDtypeStruct((_S * 8,), jnp.float32),
    ],
    scratch_types=[
        pltpu.VMEM((_CH * _E,), jnp.int32),
        pltpu.VMEM((_CH * _E,), jnp.float32),
        pltpu.VMEM((_CH * 8 + 8,), jnp.int32),
        pltpu.VMEM((_CH * 8 + 8,), jnp.float32),
    ],
)(_route_sc_body)


def kernel(hidden_states, gate_conv_w, bias):
    wpack = jnp.transpose(gate_conv_w, (1, 2, 0)).reshape(_H, _KERNEL * _E)
    bias2 = bias.reshape(1, _E).astype(jnp.float32)

    idxs, wts = [], []
    for b in range(_B):
        logits = _tc_logits(hidden_states, wpack, bias2, b).reshape(_S * _E)
        oi, ow = _route_sc(
            jax.lax.bitcast_convert_type(logits, jnp.int32), logits)
        idxs.append(oi.reshape(_S, _TOP_K))
        wts.append(ow.reshape(_S, _TOP_K))
    return jnp.stack(idxs), jnp.stack(wts)


# speedup vs baseline: 1.5018x; 1.5018x over previous
import jax
import jax.numpy as jnp
from jax.experimental import pallas as pl
from jax.experimental.pallas import tpu as pltpu

_B, _S, _H = 4, 8192, 4096
_E = 64
_TOP_K = 8
_KERNEL = 4
_BS = 1024
_NJ = _S // _BS
_NG = _B * _NJ


def _router_body(x_ref, w_ref, b_ref, idx_ref, wt_ref, ytail, ybuf):
    g = pl.program_id(0)

    @pl.when(g % _NJ == 1)
    def _():
        ytail[...] = jnp.zeros_like(ytail)

    @pl.when(g >= 1)
    def _():
        y = ybuf[...]
        prev = ytail[0:3, :]
        ycat = jnp.concatenate([prev, y], axis=0)
        logits = (ycat[3:3 + _BS, 3 * _E:4 * _E]
                  + ycat[2:2 + _BS, 2 * _E:3 * _E]
                  + ycat[1:1 + _BS, 1 * _E:2 * _E]
                  + ycat[0:_BS, 0:_E]) + b_ref[0]
        ytail[0:3, :] = y[_BS - 3:_BS, :]

        m = jnp.max(logits, axis=1, keepdims=True)
        e = jnp.exp(logits - m)
        z = jnp.sum(e, axis=1, keepdims=True)
        p = e / z

        iota = jax.lax.broadcasted_iota(jnp.int32, (_BS, _E), 1)
        bits = jax.lax.bitcast_convert_type(p, jnp.int32)
        keys = jax.lax.bitcast_convert_type(
            jnp.bitwise_or(jnp.bitwise_and(bits, -64), 63 - iota), jnp.float32)
        ks = []
        for _ in range(_TOP_K):
            kmax = jnp.max(keys, axis=1, keepdims=True)
            ks.append(kmax)
            keys = jnp.where(keys == kmax, -1.0, keys)
        kcat = jnp.concatenate(ks, axis=1)
        kbits = jax.lax.bitcast_convert_type(kcat, jnp.int32)
        idx_ref[0] = 63 - jnp.bitwise_and(kbits, 63)
        topv = jax.lax.bitcast_convert_type(
            jnp.bitwise_and(kbits, -64), jnp.float32)
        denom = jnp.sum(topv, axis=1, keepdims=True) + 1e-6
        wt_ref[0] = topv / denom

    @pl.when(g < _NG)
    def _():
        ybuf[...] = jnp.dot(x_ref[0], w_ref[...],
                            preferred_element_type=jnp.float32)


def kernel(hidden_states, gate_conv_w, bias):
    wpack = jnp.transpose(gate_conv_w, (1, 2, 0)).reshape(_H, _KERNEL * _E)
    bias2 = bias.reshape(1, _E).astype(jnp.float32)

    out_shape = (
        jax.ShapeDtypeStruct((_B, _S, _TOP_K), jnp.int32),
        jax.ShapeDtypeStruct((_B, _S, _TOP_K), jnp.float32),
    )

    def _xmap(g):
        gc = jnp.minimum(g, _NG - 1)
        return (gc // _NJ, gc % _NJ, 0)

    def _omap(g):
        gp = jnp.maximum(g - 1, 0)
        return (gp // _NJ, gp % _NJ, 0)

    idx, wt = pl.pallas_call(
        _router_body,
        grid=(_NG + 1,),
        in_specs=[
            pl.BlockSpec((1, _BS, _H), _xmap),
            pl.BlockSpec((_H, _KERNEL * _E), lambda g: (0, 0)),
            pl.BlockSpec((1, _E), lambda g: (0, 0)),
        ],
        out_specs=(
            pl.BlockSpec((1, _BS, _TOP_K), _omap),
            pl.BlockSpec((1, _BS, _TOP_K), _omap),
        ),
        out_shape=out_shape,
        scratch_shapes=[
            pltpu.VMEM((8, _KERNEL * _E), jnp.float32),
            pltpu.VMEM((_BS, _KERNEL * _E), jnp.float32),
        ],
        compiler_params=pltpu.CompilerParams(
            dimension_semantics=("arbitrary",),
        ),
    )(hidden_states, wpack, bias2)
    return idx, wt
